# baseline (speedup 1.0000x reference)
import functools

import jax
import jax.numpy as jnp
from jax import lax
from jax.experimental import pallas as pl
from jax.experimental.pallas import tpu as pltpu
from jax.experimental.pallas import tpu_sc as plsc

_N = 10000
_NP = 10240
_E = 320000
_EP = 327680
_CHUNK = 128
_NCHUNKS = _EP // _CHUNK
_NTILES = 32
_K = _NCHUNKS // _NTILES
_RPT = _NP // 16
_BS = 2048


_NBUF = 4


def _ring_loop(ghalf, gidx_v, sidx_v, bufs, accum, gsems, ssems, K):
    def gwait(b, j):
        pltpu.make_async_copy(ghalf.at[gidx_v.at[j]], bufs[b], gsems[b]).wait()

    def swait(b, j):
        pltpu.make_async_copy(bufs[b], accum.at[sidx_v.at[j]], ssems[b]).wait()

    for b in range(_NBUF):
        pltpu.async_copy(ghalf.at[gidx_v.at[b]], bufs[b], gsems[b])

    def body(q, carry):
        for b in range(_NBUF):
            j = _NBUF * q + b
            gwait(b, j)
            pltpu.async_copy(bufs[b], accum.at[sidx_v.at[j]], ssems[b], add=True)
        for b in range(_NBUF):
            j = _NBUF * q + b
            swait(b, j)
            pltpu.async_copy(ghalf.at[gidx_v.at[j + _NBUF]], bufs[b], gsems[b])
        return carry

    lax.fori_loop(0, K // _NBUF - 1, body, 0)
    jlast = K - _NBUF
    for b in range(_NBUF):
        gwait(b, jlast + b)
        pltpu.async_copy(bufs[b], accum.at[sidx_v.at[jlast + b]], ssems[b],
                         add=True)
    for b in range(_NBUF):
        swait(b, jlast + b)


def _spass(D, const_source):
    mesh = plsc.VectorSubcoreMesh(core_axis_name="c", subcore_axis_name="s")
    scratch = [
        pltpu.VMEM((_K, _CHUNK), jnp.int32),
        pltpu.VMEM((_K, _CHUNK), jnp.int32),
        [pltpu.VMEM((_CHUNK, D), jnp.float32) for _ in range(_NBUF)],
        pltpu.VMEM_SHARED((_NP, D), jnp.float32),
        [pltpu.SemaphoreType.DMA for _ in range(_NBUF)],
        [pltpu.SemaphoreType.DMA for _ in range(_NBUF)],
    ]

    @functools.partial(
        pl.kernel,
        out_type=jax.ShapeDtypeStruct((2, _NP, D), jnp.float32),
        mesh=mesh,
        scratch_types=scratch,
        compiler_params=pltpu.CompilerParams(use_tc_tiling_on_sc=(D % 128 == 0)),
        name=f"spass_d{D}{'_const' if const_source else ''}",
    )
    def k(g_hbm, gidx_hbm, sidx_hbm, zeros_hbm, out_hbm,
          gidx_v, sidx_v, bufs, accum, gsems, ssems):
        c = lax.axis_index("c")
        s = lax.axis_index("s")
        wid = c * 16 + s
        pltpu.sync_copy(sidx_hbm.at[pl.ds(wid * _K, _K)], sidx_v)
        if not const_source:
            pltpu.sync_copy(gidx_hbm.at[pl.ds(wid * _K, _K)], gidx_v)
        pltpu.sync_copy(zeros_hbm, accum.at[pl.ds(s * _RPT, _RPT)])
        if const_source:
            pltpu.sync_copy(g_hbm, bufs[0])
        plsc.subcore_barrier()

        if const_source:
            pltpu.async_copy(bufs[0], accum.at[sidx_v.at[0]], ssems[0], add=True)
            pltpu.async_copy(bufs[0], accum.at[sidx_v.at[1]], ssems[1], add=True)

            def body(p, carry):
                pltpu.make_async_copy(bufs[0], accum.at[sidx_v.at[2 * p]],
                                      ssems[0]).wait()
                pltpu.async_copy(bufs[0], accum.at[sidx_v.at[2 * p + 2]],
                                 ssems[0], add=True)
                pltpu.make_async_copy(bufs[0], accum.at[sidx_v.at[2 * p + 1]],
                                      ssems[1]).wait()
                pltpu.async_copy(bufs[0], accum.at[sidx_v.at[2 * p + 3]],
                                 ssems[1], add=True)
                return carry

            lax.fori_loop(0, _K // 2 - 1, body, 0)
            pltpu.make_async_copy(bufs[0], accum.at[sidx_v.at[_K - 2]],
                                  ssems[0]).wait()
            pltpu.make_async_copy(bufs[0], accum.at[sidx_v.at[_K - 1]],
                                  ssems[1]).wait()
        else:
            _ring_loop(g_hbm, gidx_v, sidx_v, bufs, accum, gsems, ssems, _K)

        plsc.subcore_barrier()
        pltpu.sync_copy(accum.at[pl.ds(s * _RPT, _RPT)],
                        out_hbm.at[c].at[pl.ds(s * _RPT, _RPT)])

    return k


def _l2_fused():
    K2 = _NCHUNKS // 16
    mesh = plsc.VectorSubcoreMesh(core_axis_name="c", subcore_axis_name="s")
    half = jax.ShapeDtypeStruct((2, _NP, 64), jnp.float32)
    scratch = [
        pltpu.VMEM((K2, _CHUNK), jnp.int32),
        pltpu.VMEM((K2, _CHUNK), jnp.int32),
        [pltpu.VMEM((_CHUNK, 64), jnp.float32) for _ in range(_NBUF)],
        pltpu.VMEM((_CHUNK, 16), jnp.float32),
        pltpu.VMEM_SHARED((_NP, 64), jnp.float32),
        [pltpu.SemaphoreType.DMA for _ in range(_NBUF)],
        [pltpu.SemaphoreType.DMA for _ in range(_NBUF)],
    ]

    @functools.partial(
        pl.kernel,
        out_type=(half, half, half),
        mesh=mesh,
        scratch_types=scratch,
        compiler_params=pltpu.CompilerParams(use_tc_tiling_on_sc=False),
        name="l2_fused128",
    )
    def k(g_hbm, gidx_hbm, sidx_hbm, zeros_hbm, dis_hbm,
          p1_hbm, g2_hbm, p2_hbm,
          gidx_v, sidx_v, bufs, dbuf, accum, gsems, ssems):
        c = lax.axis_index("c")
        s = lax.axis_index("s")
        pltpu.sync_copy(sidx_hbm.at[pl.ds(s * K2, K2)], sidx_v)
        pltpu.sync_copy(gidx_hbm.at[pl.ds(s * K2, K2)], gidx_v)
        pltpu.sync_copy(zeros_hbm, accum.at[pl.ds(s * _RPT, _RPT)])
        plsc.subcore_barrier()

        _ring_loop(g_hbm.at[c], gidx_v, sidx_v, bufs, accum, gsems, ssems, K2)
        plsc.subcore_barrier()

        for sb in range(_RPT // _CHUNK):
            base = s * _RPT + sb * _CHUNK
            pltpu.sync_copy(accum.at[pl.ds(base, _CHUNK)], bufs[0])
            pltpu.sync_copy(dis_hbm.at[pl.ds(base, _CHUNK)], dbuf)

            def ew1(r, carry):
                d = dbuf[r, pl.ds(0, 16)]
                for q in range(4):
                    u = bufs[0][r, pl.ds(16 * q, 16)]
                    bufs[3][r, pl.ds(16 * q, 16)] = -(d * u)
                    bufs[1][r, pl.ds(16 * q, 16)] = -((d * d) * u)
                return carry

            lax.fori_loop(0, _CHUNK, ew1, 0)
            pltpu.sync_copy(bufs[3], p1_hbm.at[c].at[pl.ds(base, _CHUNK)])
            pltpu.sync_copy(bufs[1], g2_hbm.at[c].at[pl.ds(base, _CHUNK)])

        pltpu.sync_copy(zeros_hbm, accum.at[pl.ds(s * _RPT, _RPT)])
        plsc.subcore_barrier()

        _ring_loop(g2_hbm.at[c], gidx_v, sidx_v, bufs, accum, gsems, ssems, K2)
        plsc.subcore_barrier()

        for sb in range(_RPT // _CHUNK):
            base = s * _RPT + sb * _CHUNK
            pltpu.sync_copy(accum.at[pl.ds(base, _CHUNK)], bufs[0])
            pltpu.sync_copy(dis_hbm.at[pl.ds(base, _CHUNK)], dbuf)

            def ew2(r, carry):
                d = dbuf[r, pl.ds(0, 16)]
                for q in range(4):
                    v = bufs[0][r, pl.ds(16 * q, 16)]
                    bufs[1][r, pl.ds(16 * q, 16)] = -2.0 * (d * v)
                return carry

            lax.fori_loop(0, _CHUNK, ew2, 0)
            pltpu.sync_copy(bufs[1], p2_hbm.at[c].at[pl.ds(base, _CHUNK)])

    return k


_SP16 = _spass(16, False)
_SP16C = _spass(16, True)
_L2F = _l2_fused()


def _row(bs, w):
    return pl.BlockSpec((bs, w), lambda i: (i, 0))


def _parts(bs, w):
    return pl.BlockSpec((2, bs, w), lambda i: (0, i, 0))


def _full(shape):
    return pl.BlockSpec(shape, lambda i: tuple(0 for _ in shape))


def _tc(body, in_specs, out_specs, out_shapes):
    return pl.pallas_call(
        body,
        grid=(_NP // _BS,),
        in_specs=in_specs,
        out_specs=out_specs,
        out_shape=out_shapes,
    )


def _prep_body(degp, xpad, dis_o, g_o):
    deg = degp[0] + degp[1]
    dis = jnp.where(deg > 0, lax.rsqrt(jnp.maximum(deg, 1.0)), 0.0)
    dis_o[...] = dis
    g_o[...] = dis * xpad[...]


def _scale16_body(parts, dis, sum_o, g2_o):
    u = parts[0] + parts[1]
    d = dis[...]
    sum_o[...] = u
    g2_o[...] = -(d * d) * u


def _l1_body(usum, vparts, dis, xpad, w0, w1, w2, b, h1_o, g_o):
    d = dis[...]
    xv = xpad[...]
    tx1 = -d * usum[...]
    tx2 = -2.0 * d * (vparts[0] + vparts[1]) - xv
    c = (jnp.dot(xv, w0[...], preferred_element_type=jnp.float32)
         + jnp.dot(tx1, w1[...], preferred_element_type=jnp.float32)
         + jnp.dot(tx2, w2[...], preferred_element_type=jnp.float32)
         + b[...])
    z = c[:, :128]
    hh = c[:, 128:]
    h1 = jax.nn.relu((1.0 - jax.nn.sigmoid(z)) * jnp.tanh(hh))
    h1_o[...] = h1
    g = d[:, 0:1] * h1
    g_o[0] = g[:, :64]
    g_o[1] = g[:, 64:]


def _l2_body(h1, p1h, p2h, dis, w0c, w1c, w2c, b, wy0, wy1, wy2,
             d03_o, y1_o, gy2_o):
    p1 = jnp.concatenate([p1h[0], p1h[1]], axis=1)
    p2 = jnp.concatenate([p2h[0], p2h[1]], axis=1)
    c = (jnp.dot(h1[...], w0c[...] - w2c[...],
                 preferred_element_type=jnp.float32)
         + jnp.dot(p1, w1c[...], preferred_element_type=jnp.float32)
         + jnp.dot(p2, w2c[...], preferred_element_type=jnp.float32)
         + b[...])
    z = c[:, :256]
    hh = c[:, 256:]
    h2 = jax.nn.relu((1.0 - jax.nn.sigmoid(z)) * jnp.tanh(hh))
    y0 = jnp.dot(h2, wy0[...], preferred_element_type=jnp.float32)
    y1 = jnp.dot(h2, wy1[...], preferred_element_type=jnp.float32)
    y2 = jnp.dot(h2, wy2[...], preferred_element_type=jnp.float32)
    d03_o[...] = y0 - y2
    y1_o[...] = y1
    gy2_o[...] = dis[...] * y2


def _l3w_body(tparts, y1, dis, w_o):
    d = dis[...]
    w_o[...] = d * y1[...] - 2.0 * (d * d) * (tparts[0] + tparts[1])


def _fin_body(sparts, d03, dis, b, perm, out_o):
    cheb = d03[...] - dis[...] * (sparts[0] + sparts[1]) + b[...]
    shifted = jnp.dot(cheb, perm[...], preferred_element_type=jnp.float32)
    out_o[...] = (1.0 - jax.nn.sigmoid(cheb)) * jnp.tanh(shifted)


def kernel(x, edge_index, Wx1, Wh1, bx1, bh1, Wx2, Wh2, bx2, bh2,
           Wx3, Wh3, bx3, bh3):
    f32 = jnp.float32
    src = edge_index[0].astype(jnp.int32)
    dst = edge_index[1].astype(jnp.int32)
    pad_ids = _N + (jnp.arange(_EP - _E, dtype=jnp.int32) % (_NP - _N))
    srcp = jnp.concatenate([src, pad_ids]).reshape(_NCHUNKS, _CHUNK)
    dstp = jnp.concatenate([dst, pad_ids]).reshape(_NCHUNKS, _CHUNK)
    z16 = jnp.zeros((_RPT, 16), f32)
    z64 = jnp.zeros((_RPT, 64), f32)
    ones_blk = jnp.ones((_CHUNK, 16), f32)
    xpad = jnp.zeros((_NP, 16), f32).at[:_N, :3].set(x)

    w0p = jnp.zeros((16, 256), f32).at[:3, :128].set(Wx1[0, 0]).at[:3, 128:].set(Wx1[2, 0])
    w1p = jnp.zeros((16, 256), f32).at[:3, :128].set(Wx1[0, 1]).at[:3, 128:].set(Wx1[2, 1])
    w2p = jnp.zeros((16, 256), f32).at[:3, :128].set(Wx1[0, 2]).at[:3, 128:].set(Wx1[2, 2])
    b256 = jnp.concatenate([bx1[0] + bh1[0], bx1[2] + bh1[2]]).reshape(1, 256)
    w0c = jnp.concatenate([Wx2[0, 0], Wx2[2, 0]], axis=1)
    w1c = jnp.concatenate([Wx2[0, 1], Wx2[2, 1]], axis=1)
    w2c = jnp.concatenate([Wx2[0, 2], Wx2[2, 2]], axis=1)
    b512 = jnp.concatenate([bx2[0] + bh2[0], bx2[2] + bh2[2]]).reshape(1, 512)
    wy0 = jnp.zeros((256, 16), f32).at[:, 0:3].set(Wx3[0, 0]).at[:, 8:11].set(Wx3[2, 0])
    wy1 = jnp.zeros((256, 16), f32).at[:, 0:3].set(Wx3[0, 1]).at[:, 8:11].set(Wx3[2, 1])
    wy2 = jnp.zeros((256, 16), f32).at[:, 0:3].set(Wx3[0, 2]).at[:, 8:11].set(Wx3[2, 2])
    b16 = jnp.zeros((16,), f32).at[0:3].set(bx3[0] + bh3[0]).at[8:11].set(bx3[2] + bh3[2]).reshape(1, 16)
    perm = jnp.zeros((16, 16), f32).at[jnp.arange(8) + 8, jnp.arange(8)].set(1.0)

    degp = _SP16C(ones_blk, srcp, srcp, z16)
    dis16, g1 = _tc(
        _prep_body,
        [_parts(_BS, 16), _row(_BS, 16)],
        (_row(_BS, 16), _row(_BS, 16)),
        (jax.ShapeDtypeStruct((_NP, 16), f32),) * 2,
    )(degp, xpad)

    up = _SP16(g1, srcp, dstp, z16)
    usum, g2 = _tc(
        _scale16_body,
        [_parts(_BS, 16), _row(_BS, 16)],
        (_row(_BS, 16), _row(_BS, 16)),
        (jax.ShapeDtypeStruct((_NP, 16), f32),) * 2,
    )(up, dis16)
    vp = _SP16(g2, srcp, dstp, z16)
    h1, g128h = _tc(
        _l1_body,
        [_row(_BS, 16), _parts(_BS, 16), _row(_BS, 16), _row(_BS, 16),
         _full((16, 256)), _full((16, 256)), _full((16, 256)), _full((1, 256))],
        (_row(_BS, 128), _parts(_BS, 64)),
        (jax.ShapeDtypeStruct((_NP, 128), f32),
         jax.ShapeDtypeStruct((2, _NP, 64), f32)),
    )(usum, vp, dis16, xpad, w0p, w1p, w2p, b256)

    p1h, _g2s, p2h = _L2F(g128h, srcp, dstp, z64, dis16)
    d03, y1o, gy2 = _tc(
        _l2_body,
        [_row(_BS, 128), _parts(_BS, 64), _parts(_BS, 64), _row(_BS, 16),
         _full((128, 512)), _full((128, 512)), _full((128, 512)),
         _full((1, 512)), _full((256, 16)), _full((256, 16)), _full((256, 16))],
        (_row(_BS, 16), _row(_BS, 16), _row(_BS, 16)),
        (jax.ShapeDtypeStruct((_NP, 16), f32),) * 3,
    )(h1, p1h, p2h, dis16, w0c, w1c, w2c, b512, wy0, wy1, wy2)

    tp = _SP16(gy2, srcp, dstp, z16)
    w3 = _tc(
        _l3w_body,
        [_parts(_BS, 16), _row(_BS, 16), _row(_BS, 16)],
        _row(_BS, 16),
        jax.ShapeDtypeStruct((_NP, 16), f32),
    )(tp, y1o, dis16)
    sp = _SP16(w3, srcp, dstp, z16)
    out16 = _tc(
        _fin_body,
        [_parts(_BS, 16), _row(_BS, 16), _row(_BS, 16),
         _full((1, 16)), _full((16, 16))],
        _row(_BS, 16),
        jax.ShapeDtypeStruct((_NP, 16), f32),
    )(sp, d03, dis16, b16, perm)

    return out16[:_N, :3]


# speedup vs baseline: 1.0185x; 1.0185x over previous
import functools

import jax
import jax.numpy as jnp
from jax import lax
from jax.experimental import pallas as pl
from jax.experimental.pallas import tpu as pltpu
from jax.experimental.pallas import tpu_sc as plsc

_N = 10000
_NP = 10240
_E = 320000
_EP = 327680
_CHUNK = 128
_NCHUNKS = _EP // _CHUNK
_NTILES = 32
_K = _NCHUNKS // _NTILES
_RPT = _NP // 16
_BS = 2048


_NBUF = 4


def _ring_loop(ghalf, gidx_v, sidx_v, bufs, accum, gsems, ssems, K):
    def gwait(b, j):
        pltpu.make_async_copy(ghalf.at[gidx_v.at[j]], bufs[b], gsems[b]).wait()

    def swait(b, j):
        pltpu.make_async_copy(bufs[b], accum.at[sidx_v.at[j]], ssems[b]).wait()

    for b in range(_NBUF):
        pltpu.async_copy(ghalf.at[gidx_v.at[b]], bufs[b], gsems[b])

    def body(q, carry):
        for b in range(_NBUF):
            j = _NBUF * q + b
            gwait(b, j)
            pltpu.async_copy(bufs[b], accum.at[sidx_v.at[j]], ssems[b], add=True)
        for b in range(_NBUF):
            j = _NBUF * q + b
            swait(b, j)
            pltpu.async_copy(ghalf.at[gidx_v.at[j + _NBUF]], bufs[b], gsems[b])
        return carry

    lax.fori_loop(0, K // _NBUF - 1, body, 0)
    jlast = K - _NBUF
    for b in range(_NBUF):
        gwait(b, jlast + b)
        pltpu.async_copy(bufs[b], accum.at[sidx_v.at[jlast + b]], ssems[b],
                         add=True)
    for b in range(_NBUF):
        swait(b, jlast + b)


def _spass_scaled(use_b):
    mesh = plsc.VectorSubcoreMesh(core_axis_name="c", subcore_axis_name="s")
    scratch = [
        pltpu.VMEM((_K, _CHUNK), jnp.int32),
        pltpu.VMEM((_K, _CHUNK), jnp.int32),
        [pltpu.VMEM((_CHUNK, 16), jnp.float32) for _ in range(_NBUF)],
        pltpu.VMEM((_CHUNK, 16), jnp.float32),
        pltpu.VMEM_SHARED((_NP, 16), jnp.float32),
        [pltpu.SemaphoreType.DMA for _ in range(_NBUF)],
        [pltpu.SemaphoreType.DMA for _ in range(_NBUF)],
    ]
    gshape = jax.ShapeDtypeStruct((2, _NP, 16), jnp.float32)

    @functools.partial(
        pl.kernel,
        out_type=(jax.ShapeDtypeStruct((2, _NP, 16), jnp.float32), gshape),
        mesh=mesh,
        scratch_types=scratch,
        compiler_params=pltpu.CompilerParams(use_tc_tiling_on_sc=False),
        name=f"spass_scaled{'_b' if use_b else ''}",
    )
    def k(aparts_hbm, b_hbm, dis_hbm, gidx_hbm, sidx_hbm, zeros_hbm,
          out_hbm, gtab_hbm,
          gidx_v, sidx_v, bufs, dbuf, accum, gsems, ssems):
        c = lax.axis_index("c")
        s = lax.axis_index("s")
        wid = c * 16 + s
        pltpu.sync_copy(sidx_hbm.at[pl.ds(wid * _K, _K)], sidx_v)
        pltpu.sync_copy(gidx_hbm.at[pl.ds(wid * _K, _K)], gidx_v)
        pltpu.sync_copy(zeros_hbm, accum.at[pl.ds(s * _RPT, _RPT)])
        for sb in range(_RPT // _CHUNK):
            base = s * _RPT + sb * _CHUNK
            pltpu.sync_copy(aparts_hbm.at[0].at[pl.ds(base, _CHUNK)], bufs[0])
            pltpu.sync_copy(aparts_hbm.at[1].at[pl.ds(base, _CHUNK)], bufs[1])
            pltpu.sync_copy(dis_hbm.at[pl.ds(base, _CHUNK)], dbuf)
            if use_b:
                pltpu.sync_copy(b_hbm.at[pl.ds(base, _CHUNK)], bufs[2])

            def ew(r, carry):
                d = dbuf[r, pl.ds(0, 16)]
                a = bufs[0][r, pl.ds(0, 16)] + bufs[1][r, pl.ds(0, 16)]
                if use_b:
                    bb = bufs[2][r, pl.ds(0, 16)]
                    bufs[3][r, pl.ds(0, 16)] = d * bb - 2.0 * ((d * d) * a)
                else:
                    bufs[3][r, pl.ds(0, 16)] = -((d * d) * a)
                return carry

            lax.fori_loop(0, _CHUNK, ew, 0)
            pltpu.sync_copy(bufs[3], gtab_hbm.at[c].at[pl.ds(base, _CHUNK)])
        plsc.subcore_barrier()

        _ring_loop(gtab_hbm.at[c], gidx_v, sidx_v, bufs, accum, gsems, ssems,
                   _K)
        plsc.subcore_barrier()
        pltpu.sync_copy(accum.at[pl.ds(s * _RPT, _RPT)],
                        out_hbm.at[c].at[pl.ds(s * _RPT, _RPT)])

    return k


def _spass(D, const_source):
    mesh = plsc.VectorSubcoreMesh(core_axis_name="c", subcore_axis_name="s")
    scratch = [
        pltpu.VMEM((_K, _CHUNK), jnp.int32),
        pltpu.VMEM((_K, _CHUNK), jnp.int32),
        [pltpu.VMEM((_CHUNK, D), jnp.float32) for _ in range(_NBUF)],
        pltpu.VMEM_SHARED((_NP, D), jnp.float32),
        [pltpu.SemaphoreType.DMA for _ in range(_NBUF)],
        [pltpu.SemaphoreType.DMA for _ in range(_NBUF)],
    ]

    @functools.partial(
        pl.kernel,
        out_type=jax.ShapeDtypeStruct((2, _NP, D), jnp.float32),
        mesh=mesh,
        scratch_types=scratch,
        compiler_params=pltpu.CompilerParams(use_tc_tiling_on_sc=(D % 128 == 0)),
        name=f"spass_d{D}{'_const' if const_source else ''}",
    )
    def k(g_hbm, gidx_hbm, sidx_hbm, zeros_hbm, out_hbm,
          gidx_v, sidx_v, bufs, accum, gsems, ssems):
        c = lax.axis_index("c")
        s = lax.axis_index("s")
        wid = c * 16 + s
        pltpu.sync_copy(sidx_hbm.at[pl.ds(wid * _K, _K)], sidx_v)
        if not const_source:
            pltpu.sync_copy(gidx_hbm.at[pl.ds(wid * _K, _K)], gidx_v)
        pltpu.sync_copy(zeros_hbm, accum.at[pl.ds(s * _RPT, _RPT)])
        if const_source:
            pltpu.sync_copy(g_hbm, bufs[0])
        plsc.subcore_barrier()

        if const_source:
            pltpu.async_copy(bufs[0], accum.at[sidx_v.at[0]], ssems[0], add=True)
            pltpu.async_copy(bufs[0], accum.at[sidx_v.at[1]], ssems[1], add=True)

            def body(p, carry):
                pltpu.make_async_copy(bufs[0], accum.at[sidx_v.at[2 * p]],
                                      ssems[0]).wait()
                pltpu.async_copy(bufs[0], accum.at[sidx_v.at[2 * p + 2]],
                                 ssems[0], add=True)
                pltpu.make_async_copy(bufs[0], accum.at[sidx_v.at[2 * p + 1]],
                                      ssems[1]).wait()
                pltpu.async_copy(bufs[0], accum.at[sidx_v.at[2 * p + 3]],
                                 ssems[1], add=True)
                return carry

            lax.fori_loop(0, _K // 2 - 1, body, 0)
            pltpu.make_async_copy(bufs[0], accum.at[sidx_v.at[_K - 2]],
                                  ssems[0]).wait()
            pltpu.make_async_copy(bufs[0], accum.at[sidx_v.at[_K - 1]],
                                  ssems[1]).wait()
        else:
            _ring_loop(g_hbm, gidx_v, sidx_v, bufs, accum, gsems, ssems, _K)

        plsc.subcore_barrier()
        pltpu.sync_copy(accum.at[pl.ds(s * _RPT, _RPT)],
                        out_hbm.at[c].at[pl.ds(s * _RPT, _RPT)])

    return k


def _l2_fused():
    K2 = _NCHUNKS // 16
    mesh = plsc.VectorSubcoreMesh(core_axis_name="c", subcore_axis_name="s")
    half = jax.ShapeDtypeStruct((2, _NP, 64), jnp.float32)
    scratch = [
        pltpu.VMEM((K2, _CHUNK), jnp.int32),
        pltpu.VMEM((K2, _CHUNK), jnp.int32),
        [pltpu.VMEM((_CHUNK, 64), jnp.float32) for _ in range(_NBUF)],
        pltpu.VMEM((_CHUNK, 16), jnp.float32),
        pltpu.VMEM_SHARED((_NP, 64), jnp.float32),
        [pltpu.SemaphoreType.DMA for _ in range(_NBUF)],
        [pltpu.SemaphoreType.DMA for _ in range(_NBUF)],
    ]

    @functools.partial(
        pl.kernel,
        out_type=(half, half, half),
        mesh=mesh,
        scratch_types=scratch,
        compiler_params=pltpu.CompilerParams(use_tc_tiling_on_sc=False),
        name="l2_fused128",
    )
    def k(g_hbm, gidx_hbm, sidx_hbm, zeros_hbm, dis_hbm,
          p1_hbm, g2_hbm, p2_hbm,
          gidx_v, sidx_v, bufs, dbuf, accum, gsems, ssems):
        c = lax.axis_index("c")
        s = lax.axis_index("s")
        pltpu.sync_copy(sidx_hbm.at[pl.ds(s * K2, K2)], sidx_v)
        pltpu.sync_copy(gidx_hbm.at[pl.ds(s * K2, K2)], gidx_v)
        pltpu.sync_copy(zeros_hbm, accum.at[pl.ds(s * _RPT, _RPT)])
        plsc.subcore_barrier()

        _ring_loop(g_hbm.at[c], gidx_v, sidx_v, bufs, accum, gsems, ssems, K2)
        plsc.subcore_barrier()

        for sb in range(_RPT // _CHUNK):
            base = s * _RPT + sb * _CHUNK
            pltpu.sync_copy(accum.at[pl.ds(base, _CHUNK)], bufs[0])
            pltpu.sync_copy(dis_hbm.at[pl.ds(base, _CHUNK)], dbuf)

            def ew1(r, carry):
                d = dbuf[r, pl.ds(0, 16)]
                for q in range(4):
                    u = bufs[0][r, pl.ds(16 * q, 16)]
                    bufs[3][r, pl.ds(16 * q, 16)] = -(d * u)
                    bufs[1][r, pl.ds(16 * q, 16)] = -((d * d) * u)
                return carry

            lax.fori_loop(0, _CHUNK, ew1, 0)
            pltpu.sync_copy(bufs[3], p1_hbm.at[c].at[pl.ds(base, _CHUNK)])
            pltpu.sync_copy(bufs[1], g2_hbm.at[c].at[pl.ds(base, _CHUNK)])

        pltpu.sync_copy(zeros_hbm, accum.at[pl.ds(s * _RPT, _RPT)])
        plsc.subcore_barrier()

        _ring_loop(g2_hbm.at[c], gidx_v, sidx_v, bufs, accum, gsems, ssems, K2)
        plsc.subcore_barrier()

        for sb in range(_RPT // _CHUNK):
            base = s * _RPT + sb * _CHUNK
            pltpu.sync_copy(accum.at[pl.ds(base, _CHUNK)], bufs[0])
            pltpu.sync_copy(dis_hbm.at[pl.ds(base, _CHUNK)], dbuf)

            def ew2(r, carry):
                d = dbuf[r, pl.ds(0, 16)]
                for q in range(4):
                    v = bufs[0][r, pl.ds(16 * q, 16)]
                    bufs[1][r, pl.ds(16 * q, 16)] = -2.0 * (d * v)
                return carry

            lax.fori_loop(0, _CHUNK, ew2, 0)
            pltpu.sync_copy(bufs[1], p2_hbm.at[c].at[pl.ds(base, _CHUNK)])

    return k


_SP16 = _spass(16, False)
_SP16C = _spass(16, True)
_SP16S = _spass_scaled(False)
_SP16SB = _spass_scaled(True)
_L2F = _l2_fused()


def _row(bs, w):
    return pl.BlockSpec((bs, w), lambda i: (i, 0))


def _parts(bs, w):
    return pl.BlockSpec((2, bs, w), lambda i: (0, i, 0))


def _full(shape):
    return pl.BlockSpec(shape, lambda i: tuple(0 for _ in shape))


def _tc(body, in_specs, out_specs, out_shapes):
    return pl.pallas_call(
        body,
        grid=(_NP // _BS,),
        in_specs=in_specs,
        out_specs=out_specs,
        out_shape=out_shapes,
    )


def _prep_body(degp, xpad, dis_o, g_o):
    deg = degp[0] + degp[1]
    dis = jnp.where(deg > 0, lax.rsqrt(jnp.maximum(deg, 1.0)), 0.0)
    dis_o[...] = dis
    g_o[...] = dis * xpad[...]


def _l1_body(uparts, vparts, dis, xpad, w0, w1, w2, b, h1_o, g_o):
    d = dis[...]
    xv = xpad[...]
    tx1 = -d * (uparts[0] + uparts[1])
    tx2 = -2.0 * d * (vparts[0] + vparts[1]) - xv
    c = (jnp.dot(xv, w0[...], preferred_element_type=jnp.float32)
         + jnp.dot(tx1, w1[...], preferred_element_type=jnp.float32)
         + jnp.dot(tx2, w2[...], preferred_element_type=jnp.float32)
         + b[...])
    z = c[:, :128]
    hh = c[:, 128:]
    h1 = jax.nn.relu((1.0 - jax.nn.sigmoid(z)) * jnp.tanh(hh))
    h1_o[...] = h1
    g = d[:, 0:1] * h1
    g_o[0] = g[:, :64]
    g_o[1] = g[:, 64:]


def _l2_body(h1, p1h, p2h, dis, w0c, w1c, w2c, b, wy0, wy1, wy2,
             d03_o, y1_o, gy2_o):
    p1 = jnp.concatenate([p1h[0], p1h[1]], axis=1)
    p2 = jnp.concatenate([p2h[0], p2h[1]], axis=1)
    c = (jnp.dot(h1[...], w0c[...] - w2c[...],
                 preferred_element_type=jnp.float32)
         + jnp.dot(p1, w1c[...], preferred_element_type=jnp.float32)
         + jnp.dot(p2, w2c[...], preferred_element_type=jnp.float32)
         + b[...])
    z = c[:, :256]
    hh = c[:, 256:]
    h2 = jax.nn.relu((1.0 - jax.nn.sigmoid(z)) * jnp.tanh(hh))
    y0 = jnp.dot(h2, wy0[...], preferred_element_type=jnp.float32)
    y1 = jnp.dot(h2, wy1[...], preferred_element_type=jnp.float32)
    y2 = jnp.dot(h2, wy2[...], preferred_element_type=jnp.float32)
    d03_o[...] = y0 - y2
    y1_o[...] = y1
    gy2_o[...] = dis[...] * y2


def _fin_body(sparts, d03, dis, b, perm, out_o):
    cheb = d03[...] - dis[...] * (sparts[0] + sparts[1]) + b[...]
    shifted = jnp.dot(cheb, perm[...], preferred_element_type=jnp.float32)
    out_o[...] = (1.0 - jax.nn.sigmoid(cheb)) * jnp.tanh(shifted)


def kernel(x, edge_index, Wx1, Wh1, bx1, bh1, Wx2, Wh2, bx2, bh2,
           Wx3, Wh3, bx3, bh3):
    f32 = jnp.float32
    src = edge_index[0].astype(jnp.int32)
    dst = edge_index[1].astype(jnp.int32)
    pad_ids = _N + (jnp.arange(_EP - _E, dtype=jnp.int32) % (_NP - _N))
    srcp = jnp.concatenate([src, pad_ids]).reshape(_NCHUNKS, _CHUNK)
    dstp = jnp.concatenate([dst, pad_ids]).reshape(_NCHUNKS, _CHUNK)
    z16 = jnp.zeros((_RPT, 16), f32)
    z64 = jnp.zeros((_RPT, 64), f32)
    ones_blk = jnp.ones((_CHUNK, 16), f32)
    xpad = jnp.zeros((_NP, 16), f32).at[:_N, :3].set(x)

    w0p = jnp.zeros((16, 256), f32).at[:3, :128].set(Wx1[0, 0]).at[:3, 128:].set(Wx1[2, 0])
    w1p = jnp.zeros((16, 256), f32).at[:3, :128].set(Wx1[0, 1]).at[:3, 128:].set(Wx1[2, 1])
    w2p = jnp.zeros((16, 256), f32).at[:3, :128].set(Wx1[0, 2]).at[:3, 128:].set(Wx1[2, 2])
    b256 = jnp.concatenate([bx1[0] + bh1[0], bx1[2] + bh1[2]]).reshape(1, 256)
    w0c = jnp.concatenate([Wx2[0, 0], Wx2[2, 0]], axis=1)
    w1c = jnp.concatenate([Wx2[0, 1], Wx2[2, 1]], axis=1)
    w2c = jnp.concatenate([Wx2[0, 2], Wx2[2, 2]], axis=1)
    b512 = jnp.concatenate([bx2[0] + bh2[0], bx2[2] + bh2[2]]).reshape(1, 512)
    wy0 = jnp.zeros((256, 16), f32).at[:, 0:3].set(Wx3[0, 0]).at[:, 8:11].set(Wx3[2, 0])
    wy1 = jnp.zeros((256, 16), f32).at[:, 0:3].set(Wx3[0, 1]).at[:, 8:11].set(Wx3[2, 1])
    wy2 = jnp.zeros((256, 16), f32).at[:, 0:3].set(Wx3[0, 2]).at[:, 8:11].set(Wx3[2, 2])
    b16 = jnp.zeros((16,), f32).at[0:3].set(bx3[0] + bh3[0]).at[8:11].set(bx3[2] + bh3[2]).reshape(1, 16)
    perm = jnp.zeros((16, 16), f32).at[jnp.arange(8) + 8, jnp.arange(8)].set(1.0)

    degp = _SP16C(ones_blk, srcp, srcp, z16)
    dis16, g1 = _tc(
        _prep_body,
        [_parts(_BS, 16), _row(_BS, 16)],
        (_row(_BS, 16), _row(_BS, 16)),
        (jax.ShapeDtypeStruct((_NP, 16), f32),) * 2,
    )(degp, xpad)

    up = _SP16(g1, srcp, dstp, z16)
    vp, _g1s = _SP16S(up, dis16, dis16, srcp, dstp, z16)
    h1, g128h = _tc(
        _l1_body,
        [_parts(_BS, 16), _parts(_BS, 16), _row(_BS, 16), _row(_BS, 16),
         _full((16, 256)), _full((16, 256)), _full((16, 256)), _full((1, 256))],
        (_row(_BS, 128), _parts(_BS, 64)),
        (jax.ShapeDtypeStruct((_NP, 128), f32),
         jax.ShapeDtypeStruct((2, _NP, 64), f32)),
    )(up, vp, dis16, xpad, w0p, w1p, w2p, b256)

    p1h, _g2s, p2h = _L2F(g128h, srcp, dstp, z64, dis16)
    d03, y1o, gy2 = _tc(
        _l2_body,
        [_row(_BS, 128), _parts(_BS, 64), _parts(_BS, 64), _row(_BS, 16),
         _full((128, 512)), _full((128, 512)), _full((128, 512)),
         _full((1, 512)), _full((256, 16)), _full((256, 16)), _full((256, 16))],
        (_row(_BS, 16), _row(_BS, 16), _row(_BS, 16)),
        (jax.ShapeDtypeStruct((_NP, 16), f32),) * 3,
    )(h1, p1h, p2h, dis16, w0c, w1c, w2c, b512, wy0, wy1, wy2)

    tp = _SP16(gy2, srcp, dstp, z16)
    sp, _g3s = _SP16SB(tp, y1o, dis16, srcp, dstp, z16)
    out16 = _tc(
        _fin_body,
        [_parts(_BS, 16), _row(_BS, 16), _row(_BS, 16),
         _full((1, 16)), _full((16, 16))],
        _row(_BS, 16),
        jax.ShapeDtypeStruct((_NP, 16), f32),
    )(sp, d03, dis16, b16, perm)

    return out16[:_N, :3]


# speedup vs baseline: 1.0671x; 1.0477x over previous
import functools

import jax
import jax.numpy as jnp
from jax import lax
from jax.experimental import pallas as pl
from jax.experimental.pallas import tpu as pltpu
from jax.experimental.pallas import tpu_sc as plsc

_N = 10000
_NP = 10240
_E = 320000
_EP = 327680
_CHUNK = 128
_NCHUNKS = _EP // _CHUNK
_NTILES = 32
_K = _NCHUNKS // _NTILES
_RPT = _NP // 16
_BS = 2048


_NBUF = 8
_NBUF_L2 = 4


def _ring_loop(ghalf, gidx_v, sidx_v, bufs, accum, gsems, ssems, K, nbuf=_NBUF):
    def gwait(b, j):
        pltpu.make_async_copy(ghalf.at[gidx_v.at[j]], bufs[b], gsems[b]).wait()

    def swait(b, j):
        pltpu.make_async_copy(bufs[b], accum.at[sidx_v.at[j]], ssems[b]).wait()

    for b in range(nbuf):
        pltpu.async_copy(ghalf.at[gidx_v.at[b]], bufs[b], gsems[b])

    def body(q, carry):
        for b in range(nbuf):
            j = nbuf * q + b
            gwait(b, j)
            pltpu.async_copy(bufs[b], accum.at[sidx_v.at[j]], ssems[b], add=True)
        for b in range(nbuf):
            j = nbuf * q + b
            swait(b, j)
            pltpu.async_copy(ghalf.at[gidx_v.at[j + nbuf]], bufs[b], gsems[b])
        return carry

    lax.fori_loop(0, K // nbuf - 1, body, 0)
    jlast = K - nbuf
    for b in range(nbuf):
        gwait(b, jlast + b)
        pltpu.async_copy(bufs[b], accum.at[sidx_v.at[jlast + b]], ssems[b],
                         add=True)
    for b in range(nbuf):
        swait(b, jlast + b)


def _spass_scaled(use_b):
    mesh = plsc.VectorSubcoreMesh(core_axis_name="c", subcore_axis_name="s")
    scratch = [
        pltpu.VMEM((_K, _CHUNK), jnp.int32),
        pltpu.VMEM((_K, _CHUNK), jnp.int32),
        [pltpu.VMEM((_CHUNK, 16), jnp.float32) for _ in range(_NBUF)],
        pltpu.VMEM((_CHUNK, 16), jnp.float32),
        pltpu.VMEM_SHARED((_NP, 16), jnp.float32),
        [pltpu.SemaphoreType.DMA for _ in range(_NBUF)],
        [pltpu.SemaphoreType.DMA for _ in range(_NBUF)],
    ]
    gshape = jax.ShapeDtypeStruct((2, _NP, 16), jnp.float32)

    @functools.partial(
        pl.kernel,
        out_type=(jax.ShapeDtypeStruct((2, _NP, 16), jnp.float32), gshape),
        mesh=mesh,
        scratch_types=scratch,
        compiler_params=pltpu.CompilerParams(use_tc_tiling_on_sc=False),
        name=f"spass_scaled{'_b' if use_b else ''}",
    )
    def k(aparts_hbm, b_hbm, dis_hbm, gidx_hbm, sidx_hbm, zeros_hbm,
          out_hbm, gtab_hbm,
          gidx_v, sidx_v, bufs, dbuf, accum, gsems, ssems):
        c = lax.axis_index("c")
        s = lax.axis_index("s")
        wid = c * 16 + s
        pltpu.sync_copy(sidx_hbm.at[pl.ds(wid * _K, _K)], sidx_v)
        pltpu.sync_copy(gidx_hbm.at[pl.ds(wid * _K, _K)], gidx_v)
        pltpu.sync_copy(zeros_hbm, accum.at[pl.ds(s * _RPT, _RPT)])
        for sb in range(_RPT // _CHUNK):
            base = s * _RPT + sb * _CHUNK
            pltpu.sync_copy(aparts_hbm.at[0].at[pl.ds(base, _CHUNK)], bufs[0])
            pltpu.sync_copy(aparts_hbm.at[1].at[pl.ds(base, _CHUNK)], bufs[1])
            pltpu.sync_copy(dis_hbm.at[pl.ds(base, _CHUNK)], dbuf)
            if use_b:
                pltpu.sync_copy(b_hbm.at[pl.ds(base, _CHUNK)], bufs[2])

            def ew(r, carry):
                d = dbuf[r, pl.ds(0, 16)]
                a = bufs[0][r, pl.ds(0, 16)] + bufs[1][r, pl.ds(0, 16)]
                if use_b:
                    bb = bufs[2][r, pl.ds(0, 16)]
                    bufs[3][r, pl.ds(0, 16)] = d * bb - 2.0 * ((d * d) * a)
                else:
                    bufs[3][r, pl.ds(0, 16)] = -((d * d) * a)
                return carry

            lax.fori_loop(0, _CHUNK, ew, 0)
            pltpu.sync_copy(bufs[3], gtab_hbm.at[c].at[pl.ds(base, _CHUNK)])
        plsc.subcore_barrier()

        _ring_loop(gtab_hbm.at[c], gidx_v, sidx_v, bufs, accum, gsems, ssems,
                   _K)
        plsc.subcore_barrier()
        pltpu.sync_copy(accum.at[pl.ds(s * _RPT, _RPT)],
                        out_hbm.at[c].at[pl.ds(s * _RPT, _RPT)])

    return k


def _spass(D, const_source):
    mesh = plsc.VectorSubcoreMesh(core_axis_name="c", subcore_axis_name="s")
    scratch = [
        pltpu.VMEM((_K, _CHUNK), jnp.int32),
        pltpu.VMEM((_K, _CHUNK), jnp.int32),
        [pltpu.VMEM((_CHUNK, D), jnp.float32) for _ in range(_NBUF)],
        pltpu.VMEM_SHARED((_NP, D), jnp.float32),
        [pltpu.SemaphoreType.DMA for _ in range(_NBUF)],
        [pltpu.SemaphoreType.DMA for _ in range(_NBUF)],
    ]

    @functools.partial(
        pl.kernel,
        out_type=jax.ShapeDtypeStruct((2, _NP, D), jnp.float32),
        mesh=mesh,
        scratch_types=scratch,
        compiler_params=pltpu.CompilerParams(use_tc_tiling_on_sc=(D % 128 == 0)),
        name=f"spass_d{D}{'_const' if const_source else ''}",
    )
    def k(g_hbm, gidx_hbm, sidx_hbm, zeros_hbm, out_hbm,
          gidx_v, sidx_v, bufs, accum, gsems, ssems):
        c = lax.axis_index("c")
        s = lax.axis_index("s")
        wid = c * 16 + s
        pltpu.sync_copy(sidx_hbm.at[pl.ds(wid * _K, _K)], sidx_v)
        if not const_source:
            pltpu.sync_copy(gidx_hbm.at[pl.ds(wid * _K, _K)], gidx_v)
        pltpu.sync_copy(zeros_hbm, accum.at[pl.ds(s * _RPT, _RPT)])
        if const_source:
            pltpu.sync_copy(g_hbm, bufs[0])
        plsc.subcore_barrier()

        if const_source:
            pltpu.async_copy(bufs[0], accum.at[sidx_v.at[0]], ssems[0], add=True)
            pltpu.async_copy(bufs[0], accum.at[sidx_v.at[1]], ssems[1], add=True)

            def body(p, carry):
                pltpu.make_async_copy(bufs[0], accum.at[sidx_v.at[2 * p]],
                                      ssems[0]).wait()
                pltpu.async_copy(bufs[0], accum.at[sidx_v.at[2 * p + 2]],
                                 ssems[0], add=True)
                pltpu.make_async_copy(bufs[0], accum.at[sidx_v.at[2 * p + 1]],
                                      ssems[1]).wait()
                pltpu.async_copy(bufs[0], accum.at[sidx_v.at[2 * p + 3]],
                                 ssems[1], add=True)
                return carry

            lax.fori_loop(0, _K // 2 - 1, body, 0)
            pltpu.make_async_copy(bufs[0], accum.at[sidx_v.at[_K - 2]],
                                  ssems[0]).wait()
            pltpu.make_async_copy(bufs[0], accum.at[sidx_v.at[_K - 1]],
                                  ssems[1]).wait()
        else:
            _ring_loop(g_hbm, gidx_v, sidx_v, bufs, accum, gsems, ssems, _K)

        plsc.subcore_barrier()
        pltpu.sync_copy(accum.at[pl.ds(s * _RPT, _RPT)],
                        out_hbm.at[c].at[pl.ds(s * _RPT, _RPT)])

    return k


def _l2_fused():
    K2 = _NCHUNKS // 16
    mesh = plsc.VectorSubcoreMesh(core_axis_name="c", subcore_axis_name="s")
    half = jax.ShapeDtypeStruct((2, _NP, 64), jnp.float32)
    scratch = [
        pltpu.VMEM((K2, _CHUNK), jnp.int32),
        pltpu.VMEM((K2, _CHUNK), jnp.int32),
        [pltpu.VMEM((_CHUNK, 64), jnp.float32) for _ in range(_NBUF_L2)],
        pltpu.VMEM((_CHUNK, 16), jnp.float32),
        pltpu.VMEM_SHARED((_NP, 64), jnp.float32),
        [pltpu.SemaphoreType.DMA for _ in range(_NBUF_L2)],
        [pltpu.SemaphoreType.DMA for _ in range(_NBUF_L2)],
    ]

    @functools.partial(
        pl.kernel,
        out_type=(half, half, half),
        mesh=mesh,
        scratch_types=scratch,
        compiler_params=pltpu.CompilerParams(use_tc_tiling_on_sc=False),
        name="l2_fused128",
    )
    def k(g_hbm, gidx_hbm, sidx_hbm, zeros_hbm, dis_hbm,
          p1_hbm, g2_hbm, p2_hbm,
          gidx_v, sidx_v, bufs, dbuf, accum, gsems, ssems):
        c = lax.axis_index("c")
        s = lax.axis_index("s")
        pltpu.sync_copy(sidx_hbm.at[pl.ds(s * K2, K2)], sidx_v)
        pltpu.sync_copy(gidx_hbm.at[pl.ds(s * K2, K2)], gidx_v)
        pltpu.sync_copy(zeros_hbm, accum.at[pl.ds(s * _RPT, _RPT)])
        plsc.subcore_barrier()

        _ring_loop(g_hbm.at[c], gidx_v, sidx_v, bufs, accum, gsems, ssems,
                   K2, _NBUF_L2)
        plsc.subcore_barrier()

        for sb in range(_RPT // _CHUNK):
            base = s * _RPT + sb * _CHUNK
            pltpu.sync_copy(accum.at[pl.ds(base, _CHUNK)], bufs[0])
            pltpu.sync_copy(dis_hbm.at[pl.ds(base, _CHUNK)], dbuf)

            def ew1(r, carry):
                d = dbuf[r, pl.ds(0, 16)]
                for q in range(4):
                    u = bufs[0][r, pl.ds(16 * q, 16)]
                    bufs[3][r, pl.ds(16 * q, 16)] = -(d * u)
                    bufs[1][r, pl.ds(16 * q, 16)] = -((d * d) * u)
                return carry

            lax.fori_loop(0, _CHUNK, ew1, 0)
            pltpu.sync_copy(bufs[3], p1_hbm.at[c].at[pl.ds(base, _CHUNK)])
            pltpu.sync_copy(bufs[1], g2_hbm.at[c].at[pl.ds(base, _CHUNK)])

        pltpu.sync_copy(zeros_hbm, accum.at[pl.ds(s * _RPT, _RPT)])
        plsc.subcore_barrier()

        _ring_loop(g2_hbm.at[c], gidx_v, sidx_v, bufs, accum, gsems, ssems,
                   K2, _NBUF_L2)
        plsc.subcore_barrier()

        for sb in range(_RPT // _CHUNK):
            base = s * _RPT + sb * _CHUNK
            pltpu.sync_copy(accum.at[pl.ds(base, _CHUNK)], bufs[0])
            pltpu.sync_copy(dis_hbm.at[pl.ds(base, _CHUNK)], dbuf)

            def ew2(r, carry):
                d = dbuf[r, pl.ds(0, 16)]
                for q in range(4):
                    v = bufs[0][r, pl.ds(16 * q, 16)]
                    bufs[1][r, pl.ds(16 * q, 16)] = -2.0 * (d * v)
                return carry

            lax.fori_loop(0, _CHUNK, ew2, 0)
            pltpu.sync_copy(bufs[1], p2_hbm.at[c].at[pl.ds(base, _CHUNK)])

    return k


_SP16 = _spass(16, False)
_SP16C = _spass(16, True)
_SP16S = _spass_scaled(False)
_SP16SB = _spass_scaled(True)
_L2F = _l2_fused()


def _row(bs, w):
    return pl.BlockSpec((bs, w), lambda i: (i, 0))


def _parts(bs, w):
    return pl.BlockSpec((2, bs, w), lambda i: (0, i, 0))


def _full(shape):
    return pl.BlockSpec(shape, lambda i: tuple(0 for _ in shape))


def _tc(body, in_specs, out_specs, out_shapes):
    return pl.pallas_call(
        body,
        grid=(_NP // _BS,),
        in_specs=in_specs,
        out_specs=out_specs,
        out_shape=out_shapes,
    )


def _prep_body(degp, xpad, dis_o, g_o):
    deg = degp[0] + degp[1]
    dis = jnp.where(deg > 0, lax.rsqrt(jnp.maximum(deg, 1.0)), 0.0)
    dis_o[...] = dis
    g_o[...] = dis * xpad[...]


def _l1_body(uparts, vparts, dis, xpad, w0, w1, w2, b, h1_o, g_o):
    d = dis[...]
    xv = xpad[...]
    tx1 = -d * (uparts[0] + uparts[1])
    tx2 = -2.0 * d * (vparts[0] + vparts[1]) - xv
    c = (jnp.dot(xv, w0[...], preferred_element_type=jnp.float32)
         + jnp.dot(tx1, w1[...], preferred_element_type=jnp.float32)
         + jnp.dot(tx2, w2[...], preferred_element_type=jnp.float32)
         + b[...])
    z = c[:, :128]
    hh = c[:, 128:]
    h1 = jax.nn.relu((1.0 - jax.nn.sigmoid(z)) * jnp.tanh(hh))
    h1_o[...] = h1
    g = d[:, 0:1] * h1
    g_o[0] = g[:, :64]
    g_o[1] = g[:, 64:]


def _l2_body(h1, p1h, p2h, dis, w0c, w1c, w2c, b, wy0, wy1, wy2,
             d03_o, y1_o, gy2_o):
    p1 = jnp.concatenate([p1h[0], p1h[1]], axis=1)
    p2 = jnp.concatenate([p2h[0], p2h[1]], axis=1)
    c = (jnp.dot(h1[...], w0c[...] - w2c[...],
                 preferred_element_type=jnp.float32)
         + jnp.dot(p1, w1c[...], preferred_element_type=jnp.float32)
         + jnp.dot(p2, w2c[...], preferred_element_type=jnp.float32)
         + b[...])
    z = c[:, :256]
    hh = c[:, 256:]
    h2 = jax.nn.relu((1.0 - jax.nn.sigmoid(z)) * jnp.tanh(hh))
    y0 = jnp.dot(h2, wy0[...], preferred_element_type=jnp.float32)
    y1 = jnp.dot(h2, wy1[...], preferred_element_type=jnp.float32)
    y2 = jnp.dot(h2, wy2[...], preferred_element_type=jnp.float32)
    d03_o[...] = y0 - y2
    y1_o[...] = y1
    gy2_o[...] = dis[...] * y2


def _fin_body(sparts, d03, dis, b, perm, out_o):
    cheb = d03[...] - dis[...] * (sparts[0] + sparts[1]) + b[...]
    shifted = jnp.dot(cheb, perm[...], preferred_element_type=jnp.float32)
    out_o[...] = (1.0 - jax.nn.sigmoid(cheb)) * jnp.tanh(shifted)


def kernel(x, edge_index, Wx1, Wh1, bx1, bh1, Wx2, Wh2, bx2, bh2,
           Wx3, Wh3, bx3, bh3):
    f32 = jnp.float32
    src = edge_index[0].astype(jnp.int32)
    dst = edge_index[1].astype(jnp.int32)
    pad_ids = _N + (jnp.arange(_EP - _E, dtype=jnp.int32) % (_NP - _N))
    srcp = jnp.concatenate([src, pad_ids]).reshape(_NCHUNKS, _CHUNK)
    dstp = jnp.concatenate([dst, pad_ids]).reshape(_NCHUNKS, _CHUNK)
    z16 = jnp.zeros((_RPT, 16), f32)
    z64 = jnp.zeros((_RPT, 64), f32)
    ones_blk = jnp.ones((_CHUNK, 16), f32)
    xpad = jnp.zeros((_NP, 16), f32).at[:_N, :3].set(x)

    w0p = jnp.zeros((16, 256), f32).at[:3, :128].set(Wx1[0, 0]).at[:3, 128:].set(Wx1[2, 0])
    w1p = jnp.zeros((16, 256), f32).at[:3, :128].set(Wx1[0, 1]).at[:3, 128:].set(Wx1[2, 1])
    w2p = jnp.zeros((16, 256), f32).at[:3, :128].set(Wx1[0, 2]).at[:3, 128:].set(Wx1[2, 2])
    b256 = jnp.concatenate([bx1[0] + bh1[0], bx1[2] + bh1[2]]).reshape(1, 256)
    w0c = jnp.concatenate([Wx2[0, 0], Wx2[2, 0]], axis=1)
    w1c = jnp.concatenate([Wx2[0, 1], Wx2[2, 1]], axis=1)
    w2c = jnp.concatenate([Wx2[0, 2], Wx2[2, 2]], axis=1)
    b512 = jnp.concatenate([bx2[0] + bh2[0], bx2[2] + bh2[2]]).reshape(1, 512)
    wy0 = jnp.zeros((256, 16), f32).at[:, 0:3].set(Wx3[0, 0]).at[:, 8:11].set(Wx3[2, 0])
    wy1 = jnp.zeros((256, 16), f32).at[:, 0:3].set(Wx3[0, 1]).at[:, 8:11].set(Wx3[2, 1])
    wy2 = jnp.zeros((256, 16), f32).at[:, 0:3].set(Wx3[0, 2]).at[:, 8:11].set(Wx3[2, 2])
    b16 = jnp.zeros((16,), f32).at[0:3].set(bx3[0] + bh3[0]).at[8:11].set(bx3[2] + bh3[2]).reshape(1, 16)
    perm = jnp.zeros((16, 16), f32).at[jnp.arange(8) + 8, jnp.arange(8)].set(1.0)

    degp = _SP16C(ones_blk, srcp, srcp, z16)
    dis16, g1 = _tc(
        _prep_body,
        [_parts(_BS, 16), _row(_BS, 16)],
        (_row(_BS, 16), _row(_BS, 16)),
        (jax.ShapeDtypeStruct((_NP, 16), f32),) * 2,
    )(degp, xpad)

    up = _SP16(g1, srcp, dstp, z16)
    vp, _g1s = _SP16S(up, dis16, dis16, srcp, dstp, z16)
    h1, g128h = _tc(
        _l1_body,
        [_parts(_BS, 16), _parts(_BS, 16), _row(_BS, 16), _row(_BS, 16),
         _full((16, 256)), _full((16, 256)), _full((16, 256)), _full((1, 256))],
        (_row(_BS, 128), _parts(_BS, 64)),
        (jax.ShapeDtypeStruct((_NP, 128), f32),
         jax.ShapeDtypeStruct((2, _NP, 64), f32)),
    )(up, vp, dis16, xpad, w0p, w1p, w2p, b256)

    p1h, _g2s, p2h = _L2F(g128h, srcp, dstp, z64, dis16)
    d03, y1o, gy2 = _tc(
        _l2_body,
        [_row(_BS, 128), _parts(_BS, 64), _parts(_BS, 64), _row(_BS, 16),
         _full((128, 512)), _full((128, 512)), _full((128, 512)),
         _full((1, 512)), _full((256, 16)), _full((256, 16)), _full((256, 16))],
        (_row(_BS, 16), _row(_BS, 16), _row(_BS, 16)),
        (jax.ShapeDtypeStruct((_NP, 16), f32),) * 3,
    )(h1, p1h, p2h, dis16, w0c, w1c, w2c, b512, wy0, wy1, wy2)

    tp = _SP16(gy2, srcp, dstp, z16)
    sp, _g3s = _SP16SB(tp, y1o, dis16, srcp, dstp, z16)
    out16 = _tc(
        _fin_body,
        [_parts(_BS, 16), _row(_BS, 16), _row(_BS, 16),
         _full((1, 16)), _full((16, 16))],
        _row(_BS, 16),
        jax.ShapeDtypeStruct((_NP, 16), f32),
    )(sp, d03, dis16, b16, perm)

    return out16[:_N, :3]


# speedup vs baseline: 1.0688x; 1.0016x over previous
import functools

import jax
import jax.numpy as jnp
from jax import lax
from jax.experimental import pallas as pl
from jax.experimental.pallas import tpu as pltpu
from jax.experimental.pallas import tpu_sc as plsc

_N = 10000
_NP = 10240
_E = 320000
_EP = 327680
_CHUNK = 128
_NCHUNKS = _EP // _CHUNK
_NTILES = 32
_K = _NCHUNKS // _NTILES
_RPT = _NP // 16
_BS = 2048


_NBUF = 8
_NBUF_L2 = 4


def _ring_loop(ghalf, gidx_v, sidx_v, bufs, accum, gsems, ssems, K, nbuf=_NBUF):
    def gwait(b, j):
        pltpu.make_async_copy(ghalf.at[gidx_v.at[j]], bufs[b], gsems[b]).wait()

    def swait(b, j):
        pltpu.make_async_copy(bufs[b], accum.at[sidx_v.at[j]], ssems[b]).wait()

    for b in range(nbuf):
        pltpu.async_copy(ghalf.at[gidx_v.at[b]], bufs[b], gsems[b])

    def body(q, carry):
        for b in range(nbuf):
            j = nbuf * q + b
            gwait(b, j)
            pltpu.async_copy(bufs[b], accum.at[sidx_v.at[j]], ssems[b], add=True)
        for b in range(nbuf):
            j = nbuf * q + b
            swait(b, j)
            pltpu.async_copy(ghalf.at[gidx_v.at[j + nbuf]], bufs[b], gsems[b])
        return carry

    lax.fori_loop(0, K // nbuf - 1, body, 0)
    jlast = K - nbuf
    for b in range(nbuf):
        gwait(b, jlast + b)
        pltpu.async_copy(bufs[b], accum.at[sidx_v.at[jlast + b]], ssems[b],
                         add=True)
    for b in range(nbuf):
        swait(b, jlast + b)


def _spass_scaled(use_b):
    mesh = plsc.VectorSubcoreMesh(core_axis_name="c", subcore_axis_name="s")
    scratch = [
        pltpu.VMEM((_K, _CHUNK), jnp.int32),
        pltpu.VMEM((_K, _CHUNK), jnp.int32),
        [pltpu.VMEM((_CHUNK, 16), jnp.float32) for _ in range(_NBUF)],
        pltpu.VMEM((_CHUNK, 16), jnp.float32),
        pltpu.VMEM_SHARED((_NP, 16), jnp.float32),
        [pltpu.SemaphoreType.DMA for _ in range(_NBUF)],
        [pltpu.SemaphoreType.DMA for _ in range(_NBUF)],
    ]
    gshape = jax.ShapeDtypeStruct((2, _NP, 16), jnp.float32)

    @functools.partial(
        pl.kernel,
        out_type=(jax.ShapeDtypeStruct((2, _NP, 16), jnp.float32), gshape),
        mesh=mesh,
        scratch_types=scratch,
        compiler_params=pltpu.CompilerParams(use_tc_tiling_on_sc=False),
        name=f"spass_scaled{'_b' if use_b else ''}",
    )
    def k(aparts_hbm, b_hbm, dis_hbm, gidx_hbm, sidx_hbm, zeros_hbm,
          out_hbm, gtab_hbm,
          gidx_v, sidx_v, bufs, dbuf, accum, gsems, ssems):
        c = lax.axis_index("c")
        s = lax.axis_index("s")
        wid = c * 16 + s
        pltpu.sync_copy(sidx_hbm.at[pl.ds(wid * _K, _K)], sidx_v)
        pltpu.sync_copy(gidx_hbm.at[pl.ds(wid * _K, _K)], gidx_v)
        pltpu.sync_copy(zeros_hbm, accum.at[pl.ds(s * _RPT, _RPT)])
        for sb in range(_RPT // _CHUNK):
            base = s * _RPT + sb * _CHUNK
            pltpu.sync_copy(aparts_hbm.at[0].at[pl.ds(base, _CHUNK)], bufs[0])
            pltpu.sync_copy(aparts_hbm.at[1].at[pl.ds(base, _CHUNK)], bufs[1])
            pltpu.sync_copy(dis_hbm.at[pl.ds(base, _CHUNK)], dbuf)
            if use_b:
                pltpu.sync_copy(b_hbm.at[pl.ds(base, _CHUNK)], bufs[2])

            def ew(r, carry):
                d = dbuf[r, pl.ds(0, 16)]
                a = bufs[0][r, pl.ds(0, 16)] + bufs[1][r, pl.ds(0, 16)]
                if use_b:
                    bb = bufs[2][r, pl.ds(0, 16)]
                    bufs[3][r, pl.ds(0, 16)] = d * bb - 2.0 * ((d * d) * a)
                else:
                    bufs[3][r, pl.ds(0, 16)] = -((d * d) * a)
                return carry

            lax.fori_loop(0, _CHUNK, ew, 0)
            pltpu.sync_copy(bufs[3], gtab_hbm.at[c].at[pl.ds(base, _CHUNK)])
        plsc.subcore_barrier()

        _ring_loop(gtab_hbm.at[c], gidx_v, sidx_v, bufs, accum, gsems, ssems,
                   _K)
        plsc.subcore_barrier()
        pltpu.sync_copy(accum.at[pl.ds(s * _RPT, _RPT)],
                        out_hbm.at[c].at[pl.ds(s * _RPT, _RPT)])

    return k


def _spass(D, const_source):
    mesh = plsc.VectorSubcoreMesh(core_axis_name="c", subcore_axis_name="s")
    scratch = [
        pltpu.VMEM((_K, _CHUNK), jnp.int32),
        pltpu.VMEM((_K, _CHUNK), jnp.int32),
        [pltpu.VMEM((_CHUNK, D), jnp.float32) for _ in range(_NBUF)],
        pltpu.VMEM_SHARED((_NP, D), jnp.float32),
        [pltpu.SemaphoreType.DMA for _ in range(_NBUF)],
        [pltpu.SemaphoreType.DMA for _ in range(_NBUF)],
    ]

    @functools.partial(
        pl.kernel,
        out_type=jax.ShapeDtypeStruct((2, _NP, D), jnp.float32),
        mesh=mesh,
        scratch_types=scratch,
        compiler_params=pltpu.CompilerParams(use_tc_tiling_on_sc=(D % 128 == 0)),
        name=f"spass_d{D}{'_const' if const_source else ''}",
    )
    def k(g_hbm, gidx_hbm, sidx_hbm, zeros_hbm, out_hbm,
          gidx_v, sidx_v, bufs, accum, gsems, ssems):
        c = lax.axis_index("c")
        s = lax.axis_index("s")
        wid = c * 16 + s
        pltpu.sync_copy(sidx_hbm.at[pl.ds(wid * _K, _K)], sidx_v)
        if not const_source:
            pltpu.sync_copy(gidx_hbm.at[pl.ds(wid * _K, _K)], gidx_v)
        pltpu.sync_copy(zeros_hbm, accum.at[pl.ds(s * _RPT, _RPT)])
        if const_source:
            pltpu.sync_copy(g_hbm, bufs[0])
        plsc.subcore_barrier()

        if const_source:
            pltpu.async_copy(bufs[0], accum.at[sidx_v.at[0]], ssems[0], add=True)
            pltpu.async_copy(bufs[0], accum.at[sidx_v.at[1]], ssems[1], add=True)

            def body(p, carry):
                pltpu.make_async_copy(bufs[0], accum.at[sidx_v.at[2 * p]],
                                      ssems[0]).wait()
                pltpu.async_copy(bufs[0], accum.at[sidx_v.at[2 * p + 2]],
                                 ssems[0], add=True)
                pltpu.make_async_copy(bufs[0], accum.at[sidx_v.at[2 * p + 1]],
                                      ssems[1]).wait()
                pltpu.async_copy(bufs[0], accum.at[sidx_v.at[2 * p + 3]],
                                 ssems[1], add=True)
                return carry

            lax.fori_loop(0, _K // 2 - 1, body, 0)
            pltpu.make_async_copy(bufs[0], accum.at[sidx_v.at[_K - 2]],
                                  ssems[0]).wait()
            pltpu.make_async_copy(bufs[0], accum.at[sidx_v.at[_K - 1]],
                                  ssems[1]).wait()
        else:
            _ring_loop(g_hbm, gidx_v, sidx_v, bufs, accum, gsems, ssems, _K)

        plsc.subcore_barrier()
        pltpu.sync_copy(accum.at[pl.ds(s * _RPT, _RPT)],
                        out_hbm.at[c].at[pl.ds(s * _RPT, _RPT)])

    return k


def _l2_fused():
    K2 = _NCHUNKS // 16
    mesh = plsc.VectorSubcoreMesh(core_axis_name="c", subcore_axis_name="s")
    half = jax.ShapeDtypeStruct((2, _NP, 64), jnp.float32)
    scratch = [
        pltpu.VMEM((K2, _CHUNK), jnp.int32),
        pltpu.VMEM((K2, _CHUNK), jnp.int32),
        [pltpu.VMEM((_CHUNK, 64), jnp.float32) for _ in range(_NBUF_L2)],
        pltpu.VMEM((_CHUNK, 16), jnp.float32),
        pltpu.VMEM_SHARED((_NP, 64), jnp.float32),
        [pltpu.SemaphoreType.DMA for _ in range(_NBUF_L2)],
        [pltpu.SemaphoreType.DMA for _ in range(_NBUF_L2)],
    ]

    @functools.partial(
        pl.kernel,
        out_type=(half, half, half),
        mesh=mesh,
        scratch_types=scratch,
        compiler_params=pltpu.CompilerParams(use_tc_tiling_on_sc=False),
        name="l2_fused128",
    )
    def k(g_hbm, gidx_hbm, sidx_hbm, zeros_hbm, dis_hbm,
          p1_hbm, g2_hbm, p2_hbm,
          gidx_v, sidx_v, bufs, dbuf, accum, gsems, ssems):
        c = lax.axis_index("c")
        s = lax.axis_index("s")
        pltpu.sync_copy(sidx_hbm.at[pl.ds(s * K2, K2)], sidx_v)
        pltpu.sync_copy(gidx_hbm.at[pl.ds(s * K2, K2)], gidx_v)
        pltpu.sync_copy(zeros_hbm, accum.at[pl.ds(s * _RPT, _RPT)])
        plsc.subcore_barrier()

        _ring_loop(g_hbm.at[c], gidx_v, sidx_v, bufs, accum, gsems, ssems,
                   K2, _NBUF_L2)
        plsc.subcore_barrier()

        for sb in range(_RPT // _CHUNK):
            base = s * _RPT + sb * _CHUNK
            pltpu.sync_copy(accum.at[pl.ds(base, _CHUNK)], bufs[0])
            pltpu.sync_copy(dis_hbm.at[pl.ds(base, _CHUNK)], dbuf)

            def ew1(r, carry):
                d = dbuf[r, pl.ds(0, 16)]
                for q in range(4):
                    u = bufs[0][r, pl.ds(16 * q, 16)]
                    bufs[3][r, pl.ds(16 * q, 16)] = -(d * u)
                    bufs[1][r, pl.ds(16 * q, 16)] = -((d * d) * u)
                return carry

            lax.fori_loop(0, _CHUNK, ew1, 0)
            pltpu.sync_copy(bufs[3], p1_hbm.at[c].at[pl.ds(base, _CHUNK)])
            pltpu.sync_copy(bufs[1], g2_hbm.at[c].at[pl.ds(base, _CHUNK)])

        pltpu.sync_copy(zeros_hbm, accum.at[pl.ds(s * _RPT, _RPT)])
        plsc.subcore_barrier()

        _ring_loop(g2_hbm.at[c], gidx_v, sidx_v, bufs, accum, gsems, ssems,
                   K2, _NBUF_L2)
        plsc.subcore_barrier()

        for sb in range(_RPT // _CHUNK):
            base = s * _RPT + sb * _CHUNK
            pltpu.sync_copy(accum.at[pl.ds(base, _CHUNK)], bufs[0])
            pltpu.sync_copy(dis_hbm.at[pl.ds(base, _CHUNK)], dbuf)

            def ew2(r, carry):
                d = dbuf[r, pl.ds(0, 16)]
                for q in range(4):
                    v = bufs[0][r, pl.ds(16 * q, 16)]
                    bufs[1][r, pl.ds(16 * q, 16)] = -2.0 * (d * v)
                return carry

            lax.fori_loop(0, _CHUNK, ew2, 0)
            pltpu.sync_copy(bufs[1], p2_hbm.at[c].at[pl.ds(base, _CHUNK)])

    return k


_SP16 = _spass(16, False)
_SP16C = _spass(16, True)
_SP16S = _spass_scaled(False)
_SP16SB = _spass_scaled(True)
_L2F = _l2_fused()


def _row(bs, w):
    return pl.BlockSpec((bs, w), lambda i: (i, 0))


def _parts(bs, w):
    return pl.BlockSpec((2, bs, w), lambda i: (0, i, 0))


def _full(shape):
    return pl.BlockSpec(shape, lambda i: tuple(0 for _ in shape))


def _tc(body, in_specs, out_specs, out_shapes):
    return pl.pallas_call(
        body,
        grid=(_NP // _BS,),
        in_specs=in_specs,
        out_specs=out_specs,
        out_shape=out_shapes,
    )


def _prep_body(degp, xpad, dis_o, g_o):
    deg = degp[0] + degp[1]
    dis = jnp.where(deg > 0, lax.rsqrt(jnp.maximum(deg, 1.0)), 0.0)
    dis_o[...] = dis
    g_o[...] = dis * xpad[...]


def _l1_body(uparts, vparts, dis, xpad, w0, w1, w2, b, h1_o, g_o):
    d = dis[...]
    xv = xpad[...]
    tx1 = -d * (uparts[0] + uparts[1])
    tx2 = -2.0 * d * (vparts[0] + vparts[1]) - xv
    lhs = jnp.concatenate([xv, tx1, tx2], axis=1)
    rhs = jnp.concatenate([w0[...], w1[...], w2[...]], axis=0)
    c = jnp.dot(lhs, rhs, preferred_element_type=jnp.float32) + b[...]
    z = c[:, :128]
    hh = c[:, 128:]
    h1 = jax.nn.relu((1.0 - jax.nn.sigmoid(z)) * jnp.tanh(hh))
    h1_o[...] = h1
    g = d[:, 0:1] * h1
    g_o[0] = g[:, :64]
    g_o[1] = g[:, 64:]


def _l2_body(h1, p1h, p2h, dis, w0c, w1c, w2c, b, wy0, wy1, wy2,
             d03_o, y1_o, gy2_o):
    lhs = jnp.concatenate([h1[...], p1h[0], p1h[1], p2h[0], p2h[1]], axis=1)
    rhs = jnp.concatenate([w0c[...] - w2c[...], w1c[...], w2c[...]], axis=0)
    c = jnp.dot(lhs, rhs, preferred_element_type=jnp.float32) + b[...]
    z = c[:, :256]
    hh = c[:, 256:]
    h2 = jax.nn.relu((1.0 - jax.nn.sigmoid(z)) * jnp.tanh(hh))
    wy = jnp.concatenate([wy0[...], wy1[...], wy2[...]], axis=1)
    y = jnp.dot(h2, wy, preferred_element_type=jnp.float32)
    y0 = y[:, :16]
    y1 = y[:, 16:32]
    y2 = y[:, 32:]
    d03_o[...] = y0 - y2
    y1_o[...] = y1
    gy2_o[...] = dis[...] * y2


def _fin_body(sparts, d03, dis, b, perm, out_o):
    cheb = d03[...] - dis[...] * (sparts[0] + sparts[1]) + b[...]
    shifted = jnp.dot(cheb, perm[...], preferred_element_type=jnp.float32)
    out_o[...] = (1.0 - jax.nn.sigmoid(cheb)) * jnp.tanh(shifted)


def kernel(x, edge_index, Wx1, Wh1, bx1, bh1, Wx2, Wh2, bx2, bh2,
           Wx3, Wh3, bx3, bh3):
    f32 = jnp.float32
    src = edge_index[0].astype(jnp.int32)
    dst = edge_index[1].astype(jnp.int32)
    pad_ids = _N + (jnp.arange(_EP - _E, dtype=jnp.int32) % (_NP - _N))
    srcp = jnp.concatenate([src, pad_ids]).reshape(_NCHUNKS, _CHUNK)
    dstp = jnp.concatenate([dst, pad_ids]).reshape(_NCHUNKS, _CHUNK)
    z16 = jnp.zeros((_RPT, 16), f32)
    z64 = jnp.zeros((_RPT, 64), f32)
    ones_blk = jnp.ones((_CHUNK, 16), f32)
    xpad = jnp.zeros((_NP, 16), f32).at[:_N, :3].set(x)

    w0p = jnp.zeros((16, 256), f32).at[:3, :128].set(Wx1[0, 0]).at[:3, 128:].set(Wx1[2, 0])
    w1p = jnp.zeros((16, 256), f32).at[:3, :128].set(Wx1[0, 1]).at[:3, 128:].set(Wx1[2, 1])
    w2p = jnp.zeros((16, 256), f32).at[:3, :128].set(Wx1[0, 2]).at[:3, 128:].set(Wx1[2, 2])
    b256 = jnp.concatenate([bx1[0] + bh1[0], bx1[2] + bh1[2]]).reshape(1, 256)
    w0c = jnp.concatenate([Wx2[0, 0], Wx2[2, 0]], axis=1)
    w1c = jnp.concatenate([Wx2[0, 1], Wx2[2, 1]], axis=1)
    w2c = jnp.concatenate([Wx2[0, 2], Wx2[2, 2]], axis=1)
    b512 = jnp.concatenate([bx2[0] + bh2[0], bx2[2] + bh2[2]]).reshape(1, 512)
    wy0 = jnp.zeros((256, 16), f32).at[:, 0:3].set(Wx3[0, 0]).at[:, 8:11].set(Wx3[2, 0])
    wy1 = jnp.zeros((256, 16), f32).at[:, 0:3].set(Wx3[0, 1]).at[:, 8:11].set(Wx3[2, 1])
    wy2 = jnp.zeros((256, 16), f32).at[:, 0:3].set(Wx3[0, 2]).at[:, 8:11].set(Wx3[2, 2])
    b16 = jnp.zeros((16,), f32).at[0:3].set(bx3[0] + bh3[0]).at[8:11].set(bx3[2] + bh3[2]).reshape(1, 16)
    perm = jnp.zeros((16, 16), f32).at[jnp.arange(8) + 8, jnp.arange(8)].set(1.0)

    degp = _SP16C(ones_blk, srcp, srcp, z16)
    dis16, g1 = _tc(
        _prep_body,
        [_parts(_BS, 16), _row(_BS, 16)],
        (_row(_BS, 16), _row(_BS, 16)),
        (jax.ShapeDtypeStruct((_NP, 16), f32),) * 2,
    )(degp, xpad)

    up = _SP16(g1, srcp, dstp, z16)
    vp, _g1s = _SP16S(up, dis16, dis16, srcp, dstp, z16)
    h1, g128h = _tc(
        _l1_body,
        [_parts(_BS, 16), _parts(_BS, 16), _row(_BS, 16), _row(_BS, 16),
         _full((16, 256)), _full((16, 256)), _full((16, 256)), _full((1, 256))],
        (_row(_BS, 128), _parts(_BS, 64)),
        (jax.ShapeDtypeStruct((_NP, 128), f32),
         jax.ShapeDtypeStruct((2, _NP, 64), f32)),
    )(up, vp, dis16, xpad, w0p, w1p, w2p, b256)

    p1h, _g2s, p2h = _L2F(g128h, srcp, dstp, z64, dis16)
    d03, y1o, gy2 = _tc(
        _l2_body,
        [_row(_BS, 128), _parts(_BS, 64), _parts(_BS, 64), _row(_BS, 16),
         _full((128, 512)), _full((128, 512)), _full((128, 512)),
         _full((1, 512)), _full((256, 16)), _full((256, 16)), _full((256, 16))],
        (_row(_BS, 16), _row(_BS, 16), _row(_BS, 16)),
        (jax.ShapeDtypeStruct((_NP, 16), f32),) * 3,
    )(h1, p1h, p2h, dis16, w0c, w1c, w2c, b512, wy0, wy1, wy2)

    tp = _SP16(gy2, srcp, dstp, z16)
    sp, _g3s = _SP16SB(tp, y1o, dis16, srcp, dstp, z16)
    out16 = _tc(
        _fin_body,
        [_parts(_BS, 16), _row(_BS, 16), _row(_BS, 16),
         _full((1, 16)), _full((16, 16))],
        _row(_BS, 16),
        jax.ShapeDtypeStruct((_NP, 16), f32),
    )(sp, d03, dis16, b16, perm)

    return out16[:_N, :3]
